# Initial kernel scaffold; baseline (speedup 1.0000x reference)
import jax
import jax.numpy as jnp
from jax.experimental import pallas as pl


def kernel(x, edge_index, batch, W1a, b1a, g1a, t1a, W1b, b1b, g1b, t1b, W2a, b2a, g2a, t2a, W2b, b2b, g2b, t2b, Wc1, bc1, gc1, tc1, Wc2, bc2):
    raise NotImplementedError("write your pallas kernel here")


# speedup vs baseline: 8.1811x; 8.1811x over previous
import functools

import jax
import jax.numpy as jnp
from jax import lax
from jax.experimental import pallas as pl
from jax.experimental.pallas import tpu as pltpu
from jax.experimental.pallas import tpu_sc as plsc

_N, _E, _F, _H, _G, _C = 10000, 320000, 128, 64, 64, 10
_NC, _NS = 2, 16
_NW = _NC * _NS
_EPW = _E // _NW
_K = 80
_NCHUNK = _EPW // _K
_NPAD = 10240
_RPT = _NPAD // _NS
_ZR = 128


@functools.partial(
    pl.kernel,
    mesh=plsc.VectorSubcoreMesh(core_axis_name="c", subcore_axis_name="s"),
    out_type=jax.ShapeDtypeStruct((_NC, _NPAD, _H), jnp.float32),
    compiler_params=pltpu.CompilerParams(use_tc_tiling_on_sc=False),
    scratch_types=[
        pltpu.VMEM((_NCHUNK, _K), jnp.int32),
        pltpu.VMEM((_NCHUNK, _K), jnp.int32),
        pltpu.VMEM((_K, _H), jnp.float32),
        pltpu.VMEM((_ZR, _H), jnp.float32),
        pltpu.VMEM_SHARED((_NPAD, _H), jnp.float32),
        pltpu.SemaphoreType.DMA,
    ],
)
def _edge_scatter(y_hbm, src_hbm, dst_hbm, out_hbm,
                  src_v, dst_v, rows_v, zbuf, acc, sem):
    c = lax.axis_index("c")
    s = lax.axis_index("s")
    wid = c * _NS + s

    zero16 = jnp.zeros((16,), jnp.float32)

    def zrow(r, carry):
        for c4 in range(_H // 16):
            zbuf[r, pl.ds(c4 * 16, 16)] = zero16
        return carry

    lax.fori_loop(0, _ZR, zrow, 0)
    for b in range(_RPT // _ZR):
        pltpu.sync_copy(zbuf, acc.at[pl.ds(s * _RPT + b * _ZR, _ZR)])
    plsc.subcore_barrier()

    pltpu.sync_copy(src_hbm.at[wid], src_v)
    pltpu.sync_copy(dst_hbm.at[wid], dst_v)

    def chunk(j, carry):
        pltpu.async_copy(y_hbm.at[src_v.at[j]], rows_v, sem).wait()
        pltpu.sync_copy(rows_v, acc.at[dst_v.at[j]], add=True)
        return carry

    lax.fori_loop(0, _NCHUNK, chunk, 0)
    plsc.subcore_barrier()

    pltpu.sync_copy(acc.at[pl.ds(s * _RPT, _RPT)],
                    out_hbm.at[c, pl.ds(s * _RPT, _RPT)])


def _bn_relu(h, g, t):
    m = jnp.mean(h, axis=0, keepdims=True)
    v = jnp.mean((h - m) * (h - m), axis=0, keepdims=True)
    return jnp.maximum((h - m) / jnp.sqrt(v + 1e-5) * g + t, 0.0)


def _mm_body(x_ref, w_ref, o_ref):
    o_ref[...] = jnp.dot(x_ref[...], w_ref[...],
                         preferred_element_type=jnp.float32)


def _mlp_body(y_ref, p_ref, ba_ref, ga_ref, ta_ref,
              Wb_ref, bb_ref, gb_ref, tb_ref, Wnext_ref, o_ref):
    pre = y_ref[...] + p_ref[0, :_N] + p_ref[1, :_N] + ba_ref[...]
    h = _bn_relu(pre, ga_ref[...], ta_ref[...])
    h = _bn_relu(jnp.dot(h, Wb_ref[...], preferred_element_type=jnp.float32)
                 + bb_ref[...], gb_ref[...], tb_ref[...])
    o_ref[...] = jnp.dot(h, Wnext_ref[...], preferred_element_type=jnp.float32)


def _tail_body(y_ref, q_ref, batch_ref, ba_ref, ga_ref, ta_ref,
               Wb_ref, bb_ref, gb_ref, tb_ref,
               Wc1_ref, bc1_ref, gc1_ref, tc1_ref, Wc2_ref, bc2_ref, o_ref):
    pre = y_ref[...] + q_ref[0, :_N] + q_ref[1, :_N] + ba_ref[...]
    h = _bn_relu(pre, ga_ref[...], ta_ref[...])
    h = _bn_relu(jnp.dot(h, Wb_ref[...], preferred_element_type=jnp.float32)
                 + bb_ref[...], gb_ref[...], tb_ref[...])
    gids = lax.broadcasted_iota(jnp.int32, (_G, _N), 0)
    oh = (gids == batch_ref[...]).astype(jnp.float32)
    sums = jnp.dot(oh, h, preferred_element_type=jnp.float32)
    cnt = jnp.sum(oh, axis=1, keepdims=True)
    hm = sums / jnp.maximum(cnt, 1.0)
    z = _bn_relu(jnp.dot(hm, Wc1_ref[...], preferred_element_type=jnp.float32)
                 + bc1_ref[...], gc1_ref[...], tc1_ref[...])
    z = jnp.dot(z, Wc2_ref[...], preferred_element_type=jnp.float32) + bc2_ref[...]
    zmax = jnp.max(z, axis=1, keepdims=True)
    zs = z - zmax
    o_ref[...] = zs - jnp.log(jnp.sum(jnp.exp(zs), axis=1, keepdims=True))


_mm = pl.pallas_call(
    _mm_body, out_shape=jax.ShapeDtypeStruct((_N, _H), jnp.float32))

_mlp = pl.pallas_call(
    _mlp_body, out_shape=jax.ShapeDtypeStruct((_N, _H), jnp.float32))

_tail = pl.pallas_call(
    _tail_body, out_shape=jax.ShapeDtypeStruct((_G, _C), jnp.float32))


def kernel(x, edge_index, batch, W1a, b1a, g1a, t1a, W1b, b1b, g1b, t1b,
           W2a, b2a, g2a, t2a, W2b, b2b, g2b, t2b,
           Wc1, bc1, gc1, tc1, Wc2, bc2):
    r = lambda v: v.reshape(1, -1)
    src3 = edge_index[0].reshape(_NW, _NCHUNK, _K)
    dst3 = edge_index[1].reshape(_NW, _NCHUNK, _K)

    pad = lambda v: jnp.pad(v, ((0, _NPAD - _N), (0, 0)))

    y1 = _mm(x, W1a)
    p = _edge_scatter(pad(y1), src3, dst3)
    y2 = _mlp(y1, p, r(b1a), r(g1a), r(t1a), W1b, r(b1b), r(g1b), r(t1b), W2a)
    q = _edge_scatter(pad(y2), src3, dst3)
    return _tail(y2, q, batch.reshape(1, _N), r(b2a), r(g2a), r(t2a),
                 W2b, r(b2b), r(g2b), r(t2b),
                 Wc1, r(bc1), r(gc1), r(tc1), Wc2, r(bc2))


# speedup vs baseline: 12.2618x; 1.4988x over previous
import functools

import jax
import jax.numpy as jnp
from jax import lax
from jax.experimental import pallas as pl
from jax.experimental.pallas import tpu as pltpu
from jax.experimental.pallas import tpu_sc as plsc

_N, _E, _F, _H, _G, _C = 10000, 320000, 128, 64, 64, 10
_NC, _NS = 2, 16
_NW = _NC * _NS
_EPW = _E // _NW
_K = 80
_NCHUNK = _EPW // _K
_NPAD = 10240
_RPT = _NPAD // _NS
_ZR = 128


@functools.partial(
    pl.kernel,
    mesh=plsc.VectorSubcoreMesh(core_axis_name="c", subcore_axis_name="s"),
    out_type=jax.ShapeDtypeStruct((_NC, _NPAD, _H), jnp.float32),
    compiler_params=pltpu.CompilerParams(use_tc_tiling_on_sc=False),
    scratch_types=[
        pltpu.VMEM((_NCHUNK, _K), jnp.int32),
        pltpu.VMEM((_NCHUNK, _K), jnp.int32),
        pltpu.VMEM((_K, _H), jnp.float32),
        pltpu.VMEM((_K, _H), jnp.float32),
        pltpu.VMEM((_ZR, _H), jnp.float32),
        pltpu.VMEM_SHARED((_NPAD, _H), jnp.float32),
        pltpu.SemaphoreType.DMA,
        pltpu.SemaphoreType.DMA,
    ],
)
def _edge_scatter(y_hbm, src_hbm, dst_hbm, out_hbm,
                  src_v, dst_v, rows0, rows1, zbuf, acc, sem0, sem1):
    c = lax.axis_index("c")
    s = lax.axis_index("s")
    wid = c * _NS + s

    zero16 = jnp.zeros((16,), jnp.float32)

    def zrow(r, carry):
        for c4 in range(_H // 16):
            zbuf[r, pl.ds(c4 * 16, 16)] = zero16
        return carry

    lax.fori_loop(0, _ZR, zrow, 0)
    for b in range(_RPT // _ZR):
        pltpu.sync_copy(zbuf, acc.at[pl.ds(s * _RPT + b * _ZR, _ZR)])
    plsc.subcore_barrier()

    pltpu.sync_copy(src_hbm.at[wid], src_v)
    pltpu.sync_copy(dst_hbm.at[wid], dst_v)

    def gather_start(j, buf, sem):
        pltpu.async_copy(y_hbm.at[src_v.at[j]], buf, sem)

    def gather_wait(j, buf, sem):
        pltpu.make_async_copy(y_hbm.at[src_v.at[j]], buf, sem).wait()

    gather_start(0, rows0, sem0)

    def chunk2(g, carry):
        j0 = g * 2
        gather_start(j0 + 1, rows1, sem1)
        gather_wait(j0, rows0, sem0)
        pltpu.sync_copy(rows0, acc.at[dst_v.at[j0]], add=True)
        gather_start(j0 + 2, rows0, sem0)
        gather_wait(j0 + 1, rows1, sem1)
        pltpu.sync_copy(rows1, acc.at[dst_v.at[j0 + 1]], add=True)
        return carry

    lax.fori_loop(0, (_NCHUNK - 1) // 2, chunk2, 0)
    gather_wait(_NCHUNK - 1, rows0, sem0)
    pltpu.sync_copy(rows0, acc.at[dst_v.at[_NCHUNK - 1]], add=True)
    plsc.subcore_barrier()

    pltpu.sync_copy(acc.at[pl.ds(s * _RPT, _RPT)],
                    out_hbm.at[c, pl.ds(s * _RPT, _RPT)])


def _bn_relu(h, g, t):
    m = jnp.mean(h, axis=0, keepdims=True)
    v = jnp.mean((h - m) * (h - m), axis=0, keepdims=True)
    return jnp.maximum((h - m) / jnp.sqrt(v + 1e-5) * g + t, 0.0)


def _mm_body(x_ref, w_ref, o_ref):
    o_ref[...] = jnp.dot(x_ref[...], w_ref[...],
                         preferred_element_type=jnp.float32)


def _mlp_body(y_ref, p_ref, ba_ref, ga_ref, ta_ref,
              Wb_ref, bb_ref, gb_ref, tb_ref, Wnext_ref, o_ref):
    pre = y_ref[...] + p_ref[0, :_N] + p_ref[1, :_N] + ba_ref[...]
    h = _bn_relu(pre, ga_ref[...], ta_ref[...])
    h = _bn_relu(jnp.dot(h, Wb_ref[...], preferred_element_type=jnp.float32)
                 + bb_ref[...], gb_ref[...], tb_ref[...])
    o_ref[...] = jnp.dot(h, Wnext_ref[...], preferred_element_type=jnp.float32)


def _tail_body(y_ref, q_ref, batch_ref, ba_ref, ga_ref, ta_ref,
               Wb_ref, bb_ref, gb_ref, tb_ref,
               Wc1_ref, bc1_ref, gc1_ref, tc1_ref, Wc2_ref, bc2_ref, o_ref):
    pre = y_ref[...] + q_ref[0, :_N] + q_ref[1, :_N] + ba_ref[...]
    h = _bn_relu(pre, ga_ref[...], ta_ref[...])
    h = _bn_relu(jnp.dot(h, Wb_ref[...], preferred_element_type=jnp.float32)
                 + bb_ref[...], gb_ref[...], tb_ref[...])
    gids = lax.broadcasted_iota(jnp.int32, (_G, _N), 0)
    oh = (gids == batch_ref[...]).astype(jnp.float32)
    sums = jnp.dot(oh, h, preferred_element_type=jnp.float32)
    cnt = jnp.sum(oh, axis=1, keepdims=True)
    hm = sums / jnp.maximum(cnt, 1.0)
    z = _bn_relu(jnp.dot(hm, Wc1_ref[...], preferred_element_type=jnp.float32)
                 + bc1_ref[...], gc1_ref[...], tc1_ref[...])
    z = jnp.dot(z, Wc2_ref[...], preferred_element_type=jnp.float32) + bc2_ref[...]
    zmax = jnp.max(z, axis=1, keepdims=True)
    zs = z - zmax
    o_ref[...] = zs - jnp.log(jnp.sum(jnp.exp(zs), axis=1, keepdims=True))


_mm = pl.pallas_call(
    _mm_body, out_shape=jax.ShapeDtypeStruct((_N, _H), jnp.float32))

_mlp = pl.pallas_call(
    _mlp_body, out_shape=jax.ShapeDtypeStruct((_N, _H), jnp.float32))

_tail = pl.pallas_call(
    _tail_body, out_shape=jax.ShapeDtypeStruct((_G, _C), jnp.float32))


def kernel(x, edge_index, batch, W1a, b1a, g1a, t1a, W1b, b1b, g1b, t1b,
           W2a, b2a, g2a, t2a, W2b, b2b, g2b, t2b,
           Wc1, bc1, gc1, tc1, Wc2, bc2):
    r = lambda v: v.reshape(1, -1)
    src3 = edge_index[0].reshape(_NW, _NCHUNK, _K)
    dst3 = edge_index[1].reshape(_NW, _NCHUNK, _K)

    pad = lambda v: jnp.pad(v, ((0, _NPAD - _N), (0, 0)))

    y1 = _mm(x, W1a)
    p = _edge_scatter(pad(y1), src3, dst3)
    y2 = _mlp(y1, p, r(b1a), r(g1a), r(t1a), W1b, r(b1b), r(g1b), r(t1b), W2a)
    q = _edge_scatter(pad(y2), src3, dst3)
    return _tail(y2, q, batch.reshape(1, _N), r(b2a), r(g2a), r(t2a),
                 W2b, r(b2b), r(g2b), r(t2b),
                 Wc1, r(bc1), r(gc1), r(tc1), Wc2, r(bc2))


# speedup vs baseline: 12.4374x; 1.0143x over previous
import functools

import jax
import jax.numpy as jnp
from jax import lax
from jax.experimental import pallas as pl
from jax.experimental.pallas import tpu as pltpu
from jax.experimental.pallas import tpu_sc as plsc

_N, _E, _F, _H, _G, _C = 10000, 320000, 128, 64, 64, 10
_NC, _NS = 2, 16
_NW = _NC * _NS
_EPW = _E // _NW
_K = 80
_NCHUNK = _EPW // _K
_NPAD = 10240
_RPT = _NPAD // _NS
_ZR = 128


@functools.partial(
    pl.kernel,
    mesh=plsc.VectorSubcoreMesh(core_axis_name="c", subcore_axis_name="s"),
    out_type=jax.ShapeDtypeStruct((_NC, _NPAD, _H), jnp.float32),
    compiler_params=pltpu.CompilerParams(use_tc_tiling_on_sc=False),
    scratch_types=[
        pltpu.VMEM((_NCHUNK, _K), jnp.int32),
        pltpu.VMEM((_NCHUNK, _K), jnp.int32),
        pltpu.VMEM((_K, _H), jnp.float32),
        pltpu.VMEM((_K, _H), jnp.float32),
        pltpu.VMEM((_ZR, _H), jnp.float32),
        pltpu.VMEM_SHARED((_NPAD, _H), jnp.float32),
        pltpu.VMEM_SHARED((_NPAD, _H), jnp.float32),
        pltpu.SemaphoreType.DMA,
        pltpu.SemaphoreType.DMA,
    ],
)
def _edge_scatter(y_hbm, src_hbm, dst_hbm, out_hbm,
                  src_v, dst_v, rows0, rows1, zbuf, acc, ytab, sem0, sem1):
    c = lax.axis_index("c")
    s = lax.axis_index("s")
    wid = c * _NS + s

    zero16 = jnp.zeros((16,), jnp.float32)

    def zrow(r, carry):
        for c4 in range(_H // 16):
            zbuf[r, pl.ds(c4 * 16, 16)] = zero16
        return carry

    pltpu.sync_copy(y_hbm.at[pl.ds(s * _RPT, _RPT)],
                    ytab.at[pl.ds(s * _RPT, _RPT)])

    lax.fori_loop(0, _ZR, zrow, 0)
    for b in range(_RPT // _ZR):
        pltpu.sync_copy(zbuf, acc.at[pl.ds(s * _RPT + b * _ZR, _ZR)])
    plsc.subcore_barrier()

    pltpu.sync_copy(src_hbm.at[wid], src_v)
    pltpu.sync_copy(dst_hbm.at[wid], dst_v)

    def gather_start(j, buf, sem):
        pltpu.async_copy(ytab.at[src_v.at[j]], buf, sem)

    def gather_wait(j, buf, sem):
        pltpu.make_async_copy(ytab.at[src_v.at[j]], buf, sem).wait()

    gather_start(0, rows0, sem0)

    def chunk2(g, carry):
        j0 = g * 2
        gather_start(j0 + 1, rows1, sem1)
        gather_wait(j0, rows0, sem0)
        pltpu.sync_copy(rows0, acc.at[dst_v.at[j0]], add=True)
        gather_start(j0 + 2, rows0, sem0)
        gather_wait(j0 + 1, rows1, sem1)
        pltpu.sync_copy(rows1, acc.at[dst_v.at[j0 + 1]], add=True)
        return carry

    lax.fori_loop(0, (_NCHUNK - 1) // 2, chunk2, 0)
    gather_wait(_NCHUNK - 1, rows0, sem0)
    pltpu.sync_copy(rows0, acc.at[dst_v.at[_NCHUNK - 1]], add=True)
    plsc.subcore_barrier()

    pltpu.sync_copy(acc.at[pl.ds(s * _RPT, _RPT)],
                    out_hbm.at[c, pl.ds(s * _RPT, _RPT)])


def _bn_relu(h, g, t):
    m = jnp.mean(h, axis=0, keepdims=True)
    v = jnp.mean((h - m) * (h - m), axis=0, keepdims=True)
    return jnp.maximum((h - m) / jnp.sqrt(v + 1e-5) * g + t, 0.0)


def _mm_body(x_ref, w_ref, o_ref):
    o_ref[...] = jnp.dot(x_ref[...], w_ref[...],
                         preferred_element_type=jnp.float32)


def _mlp_body(y_ref, p_ref, ba_ref, ga_ref, ta_ref,
              Wb_ref, bb_ref, gb_ref, tb_ref, Wnext_ref, o_ref):
    pre = y_ref[...] + p_ref[0, :_N] + p_ref[1, :_N] + ba_ref[...]
    h = _bn_relu(pre, ga_ref[...], ta_ref[...])
    h = _bn_relu(jnp.dot(h, Wb_ref[...], preferred_element_type=jnp.float32)
                 + bb_ref[...], gb_ref[...], tb_ref[...])
    o_ref[...] = jnp.dot(h, Wnext_ref[...], preferred_element_type=jnp.float32)


def _tail_body(y_ref, q_ref, batch_ref, ba_ref, ga_ref, ta_ref,
               Wb_ref, bb_ref, gb_ref, tb_ref,
               Wc1_ref, bc1_ref, gc1_ref, tc1_ref, Wc2_ref, bc2_ref, o_ref):
    pre = y_ref[...] + q_ref[0, :_N] + q_ref[1, :_N] + ba_ref[...]
    h = _bn_relu(pre, ga_ref[...], ta_ref[...])
    h = _bn_relu(jnp.dot(h, Wb_ref[...], preferred_element_type=jnp.float32)
                 + bb_ref[...], gb_ref[...], tb_ref[...])
    gids = lax.broadcasted_iota(jnp.int32, (_G, _N), 0)
    oh = (gids == batch_ref[...]).astype(jnp.float32)
    sums = jnp.dot(oh, h, preferred_element_type=jnp.float32)
    cnt = jnp.sum(oh, axis=1, keepdims=True)
    hm = sums / jnp.maximum(cnt, 1.0)
    z = _bn_relu(jnp.dot(hm, Wc1_ref[...], preferred_element_type=jnp.float32)
                 + bc1_ref[...], gc1_ref[...], tc1_ref[...])
    z = jnp.dot(z, Wc2_ref[...], preferred_element_type=jnp.float32) + bc2_ref[...]
    zmax = jnp.max(z, axis=1, keepdims=True)
    zs = z - zmax
    o_ref[...] = zs - jnp.log(jnp.sum(jnp.exp(zs), axis=1, keepdims=True))


_mm = pl.pallas_call(
    _mm_body, out_shape=jax.ShapeDtypeStruct((_N, _H), jnp.float32))

_mlp = pl.pallas_call(
    _mlp_body, out_shape=jax.ShapeDtypeStruct((_N, _H), jnp.float32))

_tail = pl.pallas_call(
    _tail_body, out_shape=jax.ShapeDtypeStruct((_G, _C), jnp.float32))


def kernel(x, edge_index, batch, W1a, b1a, g1a, t1a, W1b, b1b, g1b, t1b,
           W2a, b2a, g2a, t2a, W2b, b2b, g2b, t2b,
           Wc1, bc1, gc1, tc1, Wc2, bc2):
    r = lambda v: v.reshape(1, -1)
    src3 = edge_index[0].reshape(_NW, _NCHUNK, _K)
    dst3 = edge_index[1].reshape(_NW, _NCHUNK, _K)

    pad = lambda v: jnp.pad(v, ((0, _NPAD - _N), (0, 0)))

    es = jax.jit(_edge_scatter)
    y1 = _mm(x, W1a)
    p = es(pad(y1), src3, dst3)
    y2 = _mlp(y1, p, r(b1a), r(g1a), r(t1a), W1b, r(b1b), r(g1b), r(t1b), W2a)
    q = es(pad(y2), src3, dst3)
    return _tail(y2, q, batch.reshape(1, _N), r(b2a), r(g2a), r(t2a),
                 W2b, r(b2b), r(g2b), r(t2b),
                 Wc1, r(bc1), r(gc1), r(tc1), Wc2, r(bc2))
